# Initial kernel scaffold; baseline (speedup 1.0000x reference)
import jax
import jax.numpy as jnp
from jax.experimental import pallas as pl


def kernel(z, W0, b0, g0, be0, Wl1, bl1, gl, bel, Wl2, bl2, We1, be1, ge, bee, We2, be2):
    raise NotImplementedError("write your pallas kernel here")


# speedup vs baseline: 2.8849x; 2.8849x over previous
import functools

import jax
import jax.numpy as jnp
from jax.experimental import pallas as pl

D = 256
EH = 8
LEAK = 0.05
EPS = 1e-5
TM = 512


def _leaky(x):
    return jnp.where(x >= 0, x, LEAK * x)


def _dot(a, b):
    return jnp.dot(a, b, preferred_element_type=jnp.float32)


def _k1(z_ref, w0_ref, b0_ref, y_ref, st_ref):
    i = pl.program_id(0)
    y = _leaky(_dot(z_ref[...], w0_ref[...]) + b0_ref[...])
    y_ref[...] = y
    y0, y1, y2 = y[:, :D], y[:, D:2 * D], y[:, 2 * D:]
    s1 = jnp.sum(y0 + y1 + y2, axis=0, keepdims=True)
    s2 = jnp.sum(y0 * y0 + y1 * y1 + y2 * y2, axis=0, keepdims=True)
    part = jnp.concatenate([s1, s2], axis=0)

    @pl.when(i == 0)
    def _():
        st_ref[...] = jnp.zeros_like(st_ref)

    st_ref[...] += part


def _node_bn(y, st, g0, be0, n3):
    mu = st[0:1] / n3
    var = st[1:2] / n3 - mu * mu
    rs = jax.lax.rsqrt(var + EPS)
    xs = []
    for k in range(3):
        xk = y[:, k * D:(k + 1) * D]
        xs.append(_leaky(g0 * (xk - mu) * rs + be0))
    return xs


def _k2(y_ref, st_ref, g0_ref, be0_ref, wl1_ref, bl1_ref, we1_ref, be1_ref,
        hpre_ref, he_ref, lst_ref, *, n3):
    i = pl.program_id(0)
    xs = _node_bn(y_ref[...], st_ref[...], g0_ref[...], be0_ref[...], n3)
    wt, wb = wl1_ref[:D, :], wl1_ref[D:, :]
    a0 = _dot(xs[0], wt)
    a1 = _dot(xs[1], wt)
    b1 = _dot(xs[1], wb)
    b2 = _dot(xs[2], wb)
    bl1 = bl1_ref[...]
    hpre = jnp.concatenate([a0 + b1 + bl1, a1 + b2 + bl1, a0 + b2 + bl1],
                           axis=1)
    hpre_ref[...] = hpre
    et, eb = we1_ref[:D, :], we1_ref[D:, :]
    ap0 = _dot(xs[0], et)
    ap1 = _dot(xs[1], et)
    bp1 = _dot(xs[1], eb)
    bp2 = _dot(xs[2], eb)
    be1 = be1_ref[...]
    he_ref[...] = jnp.concatenate(
        [ap0 + bp1 + be1, ap0 + bp2 + be1, ap1 + bp2 + be1], axis=1)
    s1 = jnp.sum(hpre, axis=0, keepdims=True)
    s2 = jnp.sum(hpre * hpre, axis=0, keepdims=True)
    part = jnp.concatenate([s1, s2], axis=0)

    @pl.when(i == 0)
    def _():
        lst_ref[...] = jnp.zeros_like(lst_ref)

    lst_ref[...] += part


def _k3(hpre_ref, lst_ref, gl_ref, bel_ref, wl2_ref, bl2_ref, gum_ref,
        he_ref, mask_ref, est_ref, *, n):
    i = pl.program_id(0)
    st = lst_ref[...]
    mu = st[0:1] / n
    var = st[1:2] / n - mu * mu
    rs = jax.lax.rsqrt(var + EPS)
    gl, bel = gl_ref[...], bel_ref[...]
    wl2, bl2 = wl2_ref[...], bl2_ref[...]
    hpre = hpre_ref[...]
    a = []
    for l in range(3):
        sl = slice(l * D, (l + 1) * D)
        h = _leaky(gl * (hpre[:, sl] - mu[:, sl]) * rs[:, sl] + bel)
        a.append(_leaky(_dot(h, wl2) + bl2))
    a1, a2, a3 = a
    s11 = a2[:, 1:2] + a3[:, 1:2]
    ep0 = (a1[:, 0:1] + s11) / 3
    ep1 = (a2[:, 0:1] + a1[:, 1:2] + a3[:, 1:2]) / 3
    ep2 = (a3[:, 0:1] + a2[:, 1:2] + a1[:, 1:2]) / 3
    ep3 = (a1[:, 1:2] + s11) / 3
    ep = jnp.concatenate([ep0, ep1, ep2, ep3], axis=1)
    m = jnp.max(ep, axis=1, keepdims=True)
    u = jnp.exp(ep - m)
    p = u / jnp.sum(u, axis=1, keepdims=True)
    lp = jnp.log(p + 1e-4) + gum_ref[...]
    best = lp[:, 0:1]
    idx = jnp.zeros_like(best)
    for j in (1, 2, 3):
        c = lp[:, j:j + 1]
        gt = c > best
        best = jnp.where(gt, c, best)
        idx = jnp.where(gt, jnp.float32(j), idx)
    m12 = (idx != 0.0).astype(jnp.float32)
    m13 = (idx != 2.0).astype(jnp.float32)
    m23 = (idx != 1.0).astype(jnp.float32)
    mask = jnp.concatenate([m12, m13, m23], axis=1)
    mask_ref[...] = mask
    he = he_ref[...]
    e12, e13, e23 = he[:, :EH], he[:, EH:2 * EH], he[:, 2 * EH:]
    ws = m12 * e12 + m13 * e13 + m23 * e23
    wq = m12 * e12 * e12 + m13 * e13 * e13 + m23 * e23 * e23
    s1 = jnp.sum(ws, axis=0, keepdims=True)
    s2 = jnp.sum(wq, axis=0, keepdims=True)
    cnt = jnp.sum(mask, axis=0, keepdims=True)
    row0 = jnp.concatenate([s1, jnp.sum(cnt, axis=1, keepdims=True),
                            jnp.zeros((1, 7), jnp.float32)], axis=1)
    row1 = jnp.concatenate([s2, jnp.zeros((1, 8), jnp.float32)], axis=1)
    part = jnp.concatenate([row0, row1], axis=0)

    @pl.when(i == 0)
    def _():
        est_ref[...] = jnp.zeros_like(est_ref)

    est_ref[...] += part


def _k4(y_ref, st_ref, g0_ref, be0_ref, he_ref, mask_ref, est_ref,
        ge_ref, bee_ref, we2_ref, be2_ref, out_ref, *, n3):
    xs = _node_bn(y_ref[...], st_ref[...], g0_ref[...], be0_ref[...], n3)
    est = est_ref[...]
    cnt = est[0:1, EH:EH + 1]
    mu = est[0:1, :EH] / cnt
    var = est[1:2, :EH] / cnt - mu * mu
    rs = jax.lax.rsqrt(var + EPS)
    ge, bee = ge_ref[...], bee_ref[...]
    we2, be2 = we2_ref[...], be2_ref[...]
    he = he_ref[...]
    mask = mask_ref[...]
    ea = []
    for e in range(3):
        h = he[:, e * EH:(e + 1) * EH]
        eh = _leaky(ge * (h - mu) * rs + bee)
        ea.append(_leaky(_dot(eh, we2) + be2) * mask[:, e:e + 1])
    ea12, ea13, ea23 = ea
    agg0 = ea12 + ea13
    agg1 = ea12 + ea23
    agg2 = ea13 + ea23
    out_ref[...] = jnp.concatenate(
        [xs[0], agg0, xs[1], agg1, xs[2], agg2], axis=1)


def kernel(z, W0, b0, g0, be0, Wl1, bl1, gl, bel, Wl2, bl2, We1, be1, ge,
           bee, We2, be2):
    n = z.shape[0]
    in_dim = z.shape[1]
    n3 = float(3 * n)
    nf = float(n)
    gum = jax.random.gumbel(jax.random.key(1), (n, 4), jnp.float32)
    b0r = b0.reshape(1, -1)
    g0r, be0r = g0.reshape(1, -1), be0.reshape(1, -1)
    bl1r = bl1.reshape(1, -1)
    glr, belr = gl.reshape(1, -1), bel.reshape(1, -1)
    bl2r = bl2.reshape(1, -1)
    be1r = be1.reshape(1, -1)
    ger, beer = ge.reshape(1, -1), bee.reshape(1, -1)
    be2r = be2.reshape(1, -1)
    grid = (n // TM,)
    rep = lambda i: (0, 0)
    row = lambda i: (i, 0)

    y, st = pl.pallas_call(
        _k1,
        grid=grid,
        in_specs=[pl.BlockSpec((TM, in_dim), row),
                  pl.BlockSpec((in_dim, 3 * D), rep),
                  pl.BlockSpec((1, 3 * D), rep)],
        out_specs=[pl.BlockSpec((TM, 3 * D), row),
                   pl.BlockSpec((2, D), rep)],
        out_shape=[jax.ShapeDtypeStruct((n, 3 * D), jnp.float32),
                   jax.ShapeDtypeStruct((2, D), jnp.float32)],
    )(z, W0, b0r)

    hpre, he, lst = pl.pallas_call(
        functools.partial(_k2, n3=n3),
        grid=grid,
        in_specs=[pl.BlockSpec((TM, 3 * D), row),
                  pl.BlockSpec((2, D), rep),
                  pl.BlockSpec((1, D), rep),
                  pl.BlockSpec((1, D), rep),
                  pl.BlockSpec((2 * D, D), rep),
                  pl.BlockSpec((1, D), rep),
                  pl.BlockSpec((2 * D, EH), rep),
                  pl.BlockSpec((1, EH), rep)],
        out_specs=[pl.BlockSpec((TM, 3 * D), row),
                   pl.BlockSpec((TM, 3 * EH), row),
                   pl.BlockSpec((2, 3 * D), rep)],
        out_shape=[jax.ShapeDtypeStruct((n, 3 * D), jnp.float32),
                   jax.ShapeDtypeStruct((n, 3 * EH), jnp.float32),
                   jax.ShapeDtypeStruct((2, 3 * D), jnp.float32)],
    )(y, st, g0r, be0r, Wl1, bl1r, We1, be1r)

    mask, est = pl.pallas_call(
        functools.partial(_k3, n=nf),
        grid=grid,
        in_specs=[pl.BlockSpec((TM, 3 * D), row),
                  pl.BlockSpec((2, 3 * D), rep),
                  pl.BlockSpec((1, D), rep),
                  pl.BlockSpec((1, D), rep),
                  pl.BlockSpec((D, 2), rep),
                  pl.BlockSpec((1, 2), rep),
                  pl.BlockSpec((TM, 4), row),
                  pl.BlockSpec((TM, 3 * EH), row)],
        out_specs=[pl.BlockSpec((TM, 3), row),
                   pl.BlockSpec((2, 2 * EH), rep)],
        out_shape=[jax.ShapeDtypeStruct((n, 3), jnp.float32),
                   jax.ShapeDtypeStruct((2, 2 * EH), jnp.float32)],
    )(hpre, lst, glr, belr, Wl2, bl2r, gum, he)

    out = pl.pallas_call(
        functools.partial(_k4, n3=n3),
        grid=grid,
        in_specs=[pl.BlockSpec((TM, 3 * D), row),
                  pl.BlockSpec((2, D), rep),
                  pl.BlockSpec((1, D), rep),
                  pl.BlockSpec((1, D), rep),
                  pl.BlockSpec((TM, 3 * EH), row),
                  pl.BlockSpec((TM, 3), row),
                  pl.BlockSpec((2, 2 * EH), rep),
                  pl.BlockSpec((1, EH), rep),
                  pl.BlockSpec((1, EH), rep),
                  pl.BlockSpec((EH, EH), rep),
                  pl.BlockSpec((1, EH), rep)],
        out_specs=pl.BlockSpec((TM, 3 * (D + EH)), row),
        out_shape=jax.ShapeDtypeStruct((n, 3 * (D + EH)), jnp.float32),
    )(y, st, g0r, be0r, he, mask, est, ger, beer, We2, be2r)

    return out.reshape(3 * n, D + EH)


# speedup vs baseline: 2.9436x; 1.0203x over previous
import functools

import jax
import jax.numpy as jnp
from jax.experimental import pallas as pl

D = 256
EH = 8
LEAK = 0.05
EPS = 1e-5
TM = 512


def _leaky(x):
    return jnp.where(x >= 0, x, LEAK * x)


def _dot(a, b):
    return jnp.dot(a, b, preferred_element_type=jnp.float32)


def _bdot(a, b):
    return jnp.dot(a.astype(jnp.bfloat16), b,
                   preferred_element_type=jnp.float32)


def _k1(z_ref, w0_ref, b0_ref, y_ref, st_ref):
    i = pl.program_id(0)
    y = _leaky(_dot(z_ref[...], w0_ref[...]) + b0_ref[...])
    y_ref[...] = y.astype(jnp.bfloat16)
    y0, y1, y2 = y[:, :D], y[:, D:2 * D], y[:, 2 * D:]
    s1 = jnp.sum(y0 + y1 + y2, axis=0, keepdims=True)
    s2 = jnp.sum(y0 * y0 + y1 * y1 + y2 * y2, axis=0, keepdims=True)
    part = jnp.concatenate([s1, s2], axis=0)

    @pl.when(i == 0)
    def _():
        st_ref[...] = jnp.zeros_like(st_ref)

    st_ref[...] += part


def _node_bn(y, st, g0, be0, n3):
    mu = st[0:1] / n3
    var = st[1:2] / n3 - mu * mu
    rs = jax.lax.rsqrt(var + EPS)
    xs = []
    for k in range(3):
        xk = y[:, k * D:(k + 1) * D].astype(jnp.float32)
        xs.append(_leaky(g0 * (xk - mu) * rs + be0))
    return xs


def _k2(y_ref, st_ref, g0_ref, be0_ref, wl1_ref, bl1_ref, we1_ref, be1_ref,
        hpre_ref, he_ref, lst_ref, *, n3):
    i = pl.program_id(0)
    xs = _node_bn(y_ref[...], st_ref[...], g0_ref[...], be0_ref[...], n3)
    wt, wb = wl1_ref[:D, :], wl1_ref[D:, :]
    a0 = _bdot(xs[0], wt)
    a1 = _bdot(xs[1], wt)
    b1 = _bdot(xs[1], wb)
    b2 = _bdot(xs[2], wb)
    bl1 = bl1_ref[...]
    hpre = jnp.concatenate([a0 + b1 + bl1, a1 + b2 + bl1, a0 + b2 + bl1],
                           axis=1)
    hpre_ref[...] = hpre.astype(jnp.bfloat16)
    et, eb = we1_ref[:D, :], we1_ref[D:, :]
    ap0 = _bdot(xs[0], et)
    ap1 = _bdot(xs[1], et)
    bp1 = _bdot(xs[1], eb)
    bp2 = _bdot(xs[2], eb)
    be1 = be1_ref[...]
    he_ref[...] = jnp.concatenate(
        [ap0 + bp1 + be1, ap0 + bp2 + be1, ap1 + bp2 + be1], axis=1)
    s1 = jnp.sum(hpre, axis=0, keepdims=True)
    s2 = jnp.sum(hpre * hpre, axis=0, keepdims=True)
    part = jnp.concatenate([s1, s2], axis=0)

    @pl.when(i == 0)
    def _():
        lst_ref[...] = jnp.zeros_like(lst_ref)

    lst_ref[...] += part


def _k3(hpre_ref, lst_ref, gl_ref, bel_ref, wl2_ref, bl2_ref, gum_ref,
        he_ref, mask_ref, est_ref, *, n):
    i = pl.program_id(0)
    st = lst_ref[...]
    mu = st[0:1] / n
    var = st[1:2] / n - mu * mu
    rs = jax.lax.rsqrt(var + EPS)
    gl, bel = gl_ref[...], bel_ref[...]
    wl2, bl2 = wl2_ref[...], bl2_ref[...]
    hpre = hpre_ref[...]
    a = []
    for l in range(3):
        sl = slice(l * D, (l + 1) * D)
        hl = hpre[:, sl].astype(jnp.float32)
        h = _leaky(gl * (hl - mu[:, sl]) * rs[:, sl] + bel)
        a.append(_leaky(_dot(h, wl2) + bl2))
    a1, a2, a3 = a
    s11 = a2[:, 1:2] + a3[:, 1:2]
    ep0 = (a1[:, 0:1] + s11) / 3
    ep1 = (a2[:, 0:1] + a1[:, 1:2] + a3[:, 1:2]) / 3
    ep2 = (a3[:, 0:1] + a2[:, 1:2] + a1[:, 1:2]) / 3
    ep3 = (a1[:, 1:2] + s11) / 3
    ep = jnp.concatenate([ep0, ep1, ep2, ep3], axis=1)
    m = jnp.max(ep, axis=1, keepdims=True)
    u = jnp.exp(ep - m)
    p = u / jnp.sum(u, axis=1, keepdims=True)
    lp = jnp.log(p + 1e-4) + gum_ref[...]
    best = lp[:, 0:1]
    idx = jnp.zeros_like(best)
    for j in (1, 2, 3):
        c = lp[:, j:j + 1]
        gt = c > best
        best = jnp.where(gt, c, best)
        idx = jnp.where(gt, jnp.float32(j), idx)
    m12 = (idx != 0.0).astype(jnp.float32)
    m13 = (idx != 2.0).astype(jnp.float32)
    m23 = (idx != 1.0).astype(jnp.float32)
    mask = jnp.concatenate([m12, m13, m23], axis=1)
    mask_ref[...] = mask
    he = he_ref[...]
    e12, e13, e23 = he[:, :EH], he[:, EH:2 * EH], he[:, 2 * EH:]
    ws = m12 * e12 + m13 * e13 + m23 * e23
    wq = m12 * e12 * e12 + m13 * e13 * e13 + m23 * e23 * e23
    s1 = jnp.sum(ws, axis=0, keepdims=True)
    s2 = jnp.sum(wq, axis=0, keepdims=True)
    cnt = jnp.sum(mask, axis=0, keepdims=True)
    row0 = jnp.concatenate([s1, jnp.sum(cnt, axis=1, keepdims=True),
                            jnp.zeros((1, 7), jnp.float32)], axis=1)
    row1 = jnp.concatenate([s2, jnp.zeros((1, 8), jnp.float32)], axis=1)
    part = jnp.concatenate([row0, row1], axis=0)

    @pl.when(i == 0)
    def _():
        est_ref[...] = jnp.zeros_like(est_ref)

    est_ref[...] += part


def _k4(y_ref, st_ref, g0_ref, be0_ref, he_ref, mask_ref, est_ref,
        ge_ref, bee_ref, we2_ref, be2_ref, out_ref, *, n3):
    c = pl.program_id(1)
    xs = _node_bn(y_ref[...], st_ref[...], g0_ref[...], be0_ref[...], n3)
    est = est_ref[...]
    cnt = est[0:1, EH:EH + 1]
    mu = est[0:1, :EH] / cnt
    var = est[1:2, :EH] / cnt - mu * mu
    rs = jax.lax.rsqrt(var + EPS)
    ge, bee = ge_ref[...], bee_ref[...]
    we2, be2 = we2_ref[...], be2_ref[...]
    he = he_ref[...]
    mask = mask_ref[...]
    ea = []
    for e in range(3):
        h = he[:, e * EH:(e + 1) * EH]
        eh = _leaky(ge * (h - mu) * rs + bee)
        ea.append(_leaky(_dot(eh, we2) + be2) * mask[:, e:e + 1])
    ea12, ea13, ea23 = ea
    agg = [ea12 + ea13, ea12 + ea23, ea13 + ea23]
    tm = xs[0].shape[0]
    pad = jnp.zeros((tm, 128 - EH), jnp.float32)

    @pl.when(c == 0)
    def _():
        for k in range(3):
            out_ref[pl.Slice(k, tm, 3), :] = xs[k][:, :128]

    @pl.when(c == 1)
    def _():
        for k in range(3):
            out_ref[pl.Slice(k, tm, 3), :] = xs[k][:, 128:]

    @pl.when(c == 2)
    def _():
        for k in range(3):
            out_ref[pl.Slice(k, tm, 3), :] = jnp.concatenate(
                [agg[k], pad], axis=1)


def kernel(z, W0, b0, g0, be0, Wl1, bl1, gl, bel, Wl2, bl2, We1, be1, ge,
           bee, We2, be2):
    n = z.shape[0]
    in_dim = z.shape[1]
    n3 = float(3 * n)
    nf = float(n)
    gum = jax.random.gumbel(jax.random.key(1), (n, 4), jnp.float32)
    b0r = b0.reshape(1, -1)
    g0r, be0r = g0.reshape(1, -1), be0.reshape(1, -1)
    bl1r = bl1.reshape(1, -1)
    glr, belr = gl.reshape(1, -1), bel.reshape(1, -1)
    bl2r = bl2.reshape(1, -1)
    be1r = be1.reshape(1, -1)
    ger, beer = ge.reshape(1, -1), bee.reshape(1, -1)
    be2r = be2.reshape(1, -1)
    grid = (n // TM,)
    rep = lambda i: (0, 0)
    row = lambda i: (i, 0)

    y, st = pl.pallas_call(
        _k1,
        grid=grid,
        in_specs=[pl.BlockSpec((TM, in_dim), row),
                  pl.BlockSpec((in_dim, 3 * D), rep),
                  pl.BlockSpec((1, 3 * D), rep)],
        out_specs=[pl.BlockSpec((TM, 3 * D), row),
                   pl.BlockSpec((2, D), rep)],
        out_shape=[jax.ShapeDtypeStruct((n, 3 * D), jnp.bfloat16),
                   jax.ShapeDtypeStruct((2, D), jnp.float32)],
    )(z.astype(jnp.bfloat16), W0.astype(jnp.bfloat16), b0r)

    hpre, he, lst = pl.pallas_call(
        functools.partial(_k2, n3=n3),
        grid=grid,
        in_specs=[pl.BlockSpec((TM, 3 * D), row),
                  pl.BlockSpec((2, D), rep),
                  pl.BlockSpec((1, D), rep),
                  pl.BlockSpec((1, D), rep),
                  pl.BlockSpec((2 * D, D), rep),
                  pl.BlockSpec((1, D), rep),
                  pl.BlockSpec((2 * D, EH), rep),
                  pl.BlockSpec((1, EH), rep)],
        out_specs=[pl.BlockSpec((TM, 3 * D), row),
                   pl.BlockSpec((TM, 3 * EH), row),
                   pl.BlockSpec((2, 3 * D), rep)],
        out_shape=[jax.ShapeDtypeStruct((n, 3 * D), jnp.bfloat16),
                   jax.ShapeDtypeStruct((n, 3 * EH), jnp.float32),
                   jax.ShapeDtypeStruct((2, 3 * D), jnp.float32)],
    )(y, st, g0r, be0r, Wl1.astype(jnp.bfloat16), bl1r,
      We1.astype(jnp.bfloat16), be1r)

    mask, est = pl.pallas_call(
        functools.partial(_k3, n=nf),
        grid=grid,
        in_specs=[pl.BlockSpec((TM, 3 * D), row),
                  pl.BlockSpec((2, 3 * D), rep),
                  pl.BlockSpec((1, D), rep),
                  pl.BlockSpec((1, D), rep),
                  pl.BlockSpec((D, 2), rep),
                  pl.BlockSpec((1, 2), rep),
                  pl.BlockSpec((TM, 4), row),
                  pl.BlockSpec((TM, 3 * EH), row)],
        out_specs=[pl.BlockSpec((TM, 3), row),
                   pl.BlockSpec((2, 2 * EH), rep)],
        out_shape=[jax.ShapeDtypeStruct((n, 3), jnp.float32),
                   jax.ShapeDtypeStruct((2, 2 * EH), jnp.float32)],
    )(hpre, lst, glr, belr, Wl2, bl2r, gum, he)

    row2 = lambda i, c: (i, 0)
    rep2 = lambda i, c: (0, 0)
    out = pl.pallas_call(
        functools.partial(_k4, n3=n3),
        grid=(n // TM, 3),
        in_specs=[pl.BlockSpec((TM, 3 * D), row2),
                  pl.BlockSpec((2, D), rep2),
                  pl.BlockSpec((1, D), rep2),
                  pl.BlockSpec((1, D), rep2),
                  pl.BlockSpec((TM, 3 * EH), row2),
                  pl.BlockSpec((TM, 3), row2),
                  pl.BlockSpec((2, 2 * EH), rep2),
                  pl.BlockSpec((1, EH), rep2),
                  pl.BlockSpec((1, EH), rep2),
                  pl.BlockSpec((EH, EH), rep2),
                  pl.BlockSpec((1, EH), rep2)],
        out_specs=pl.BlockSpec((3 * TM, 128), lambda i, c: (i, c)),
        out_shape=jax.ShapeDtypeStruct((3 * n, D + EH), jnp.float32),
    )(y, st, g0r, be0r, he, mask, est, ger, beer, We2, be2r)

    return out


# speedup vs baseline: 4.2784x; 1.4535x over previous
import functools

import jax
import jax.numpy as jnp
import numpy as np
from jax.experimental import pallas as pl

D = 256
D3 = 3 * D
EH = 8
LEAK = 0.05
EPS = 1e-5
TM = 512

def _leaky(x):
    return jnp.where(x >= 0, x, LEAK * x)


def _dot(a, b):
    return jnp.dot(a, b, preferred_element_type=jnp.float32)


def _bdot(a, b):
    return jnp.dot(a.astype(jnp.bfloat16), b,
                   preferred_element_type=jnp.float32)


def _k1(z_ref, w0_ref, b0_ref, y_ref, st_ref):
    i = pl.program_id(0)
    zb = z_ref[...].astype(jnp.bfloat16)
    y = _leaky(_dot(zb, w0_ref[...]) + b0_ref[...])
    y_ref[...] = y.astype(jnp.bfloat16)
    y0, y1, y2 = y[:, :D], y[:, D:2 * D], y[:, 2 * D:]
    s1 = jnp.sum(y0 + y1 + y2, axis=0, keepdims=True)
    s2 = jnp.sum(y0 * y0 + y1 * y1 + y2 * y2, axis=0, keepdims=True)
    part = jnp.concatenate([s1, s2], axis=0)

    @pl.when(i == 0)
    def _():
        st_ref[...] = jnp.zeros_like(st_ref)

    st_ref[...] += part


def _bn_cols(y, st, g0, be0, n3, lo, hi):
    mu = st[0:1, lo:hi] / n3
    var = st[1:2, lo:hi] / n3 - mu * mu
    rs = jax.lax.rsqrt(var + EPS)
    g, b = g0[:, lo:hi], be0[:, lo:hi]
    out = []
    for k in range(3):
        xk = y[:, k * D + lo:k * D + hi].astype(jnp.float32)
        out.append(_leaky(g * (xk - mu) * rs + b))
    return out


def _k2(y_ref, st_ref, g0_ref, be0_ref, wl1_ref, bl1_ref, we1_ref, be1_ref,
        hpre_ref, he_ref, lst_ref, *, n3):
    i = pl.program_id(0)
    xs = _bn_cols(y_ref[...], st_ref[...], g0_ref[...], be0_ref[...],
                  n3, 0, D)
    wt, wb = wl1_ref[:D, :], wl1_ref[D:, :]
    a0 = _bdot(xs[0], wt)
    a1 = _bdot(xs[1], wt)
    b1 = _bdot(xs[1], wb)
    b2 = _bdot(xs[2], wb)
    bl1 = bl1_ref[...]
    hpre = jnp.concatenate([a0 + b1 + bl1, a1 + b2 + bl1, a0 + b2 + bl1],
                           axis=1)
    hpre_ref[...] = hpre.astype(jnp.bfloat16)
    et, eb = we1_ref[:D, :], we1_ref[D:, :]
    ap0 = _bdot(xs[0], et)
    ap1 = _bdot(xs[1], et)
    bp1 = _bdot(xs[1], eb)
    bp2 = _bdot(xs[2], eb)
    be1 = be1_ref[...]
    he_ref[...] = jnp.concatenate(
        [ap0 + bp1 + be1, ap0 + bp2 + be1, ap1 + bp2 + be1], axis=1)
    s1 = jnp.sum(hpre, axis=0, keepdims=True)
    s2 = jnp.sum(hpre * hpre, axis=0, keepdims=True)
    part = jnp.concatenate([s1, s2], axis=0)

    @pl.when(i == 0)
    def _():
        lst_ref[...] = jnp.zeros_like(lst_ref)

    lst_ref[...] += part


def _k3(hpre_ref, lst_ref, gl3_ref, bel3_ref, wl2x_ref, bl2x_ref, gum_ref,
        he_ref, maskc_ref, est_ref, *, n):
    i = pl.program_id(0)
    st = lst_ref[...]
    mu = st[0:1] / n
    var = st[1:2] / n - mu * mu
    rs = jax.lax.rsqrt(var + EPS)
    hl = hpre_ref[...].astype(jnp.float32)
    h = _leaky(gl3_ref[...] * (hl - mu) * rs + bel3_ref[...])
    a8 = _leaky(_bdot(h, wl2x_ref[...]) + bl2x_ref[...])
    at = jnp.transpose(a8)
    r = [at[j:j + 1, :] for j in range(6)]
    s135 = r[1] + r[3] + r[5]
    ep0 = (r[0] + r[3] + r[5]) / 3
    ep1 = (r[2] + r[1] + r[5]) / 3
    ep2 = (r[4] + r[3] + r[1]) / 3
    ep3 = s135 / 3
    m = jnp.maximum(jnp.maximum(ep0, ep1), jnp.maximum(ep2, ep3))
    u0 = jnp.exp(ep0 - m)
    u1 = jnp.exp(ep1 - m)
    u2 = jnp.exp(ep2 - m)
    u3 = jnp.exp(ep3 - m)
    s = u0 + u1 + u2 + u3
    gum = gum_ref[...]
    l0 = jnp.log(u0 / s + 1e-4) + gum[0:1, :]
    l1 = jnp.log(u1 / s + 1e-4) + gum[1:2, :]
    l2 = jnp.log(u2 / s + 1e-4) + gum[2:3, :]
    l3 = jnp.log(u3 / s + 1e-4) + gum[3:4, :]
    best = l0
    idx = jnp.zeros_like(l0)
    for j, lj in ((1, l1), (2, l2), (3, l3)):
        gt = lj > best
        best = jnp.where(gt, lj, best)
        idx = jnp.where(gt, jnp.float32(j), idx)
    m12 = (idx != 0.0).astype(jnp.float32)
    m13 = (idx != 2.0).astype(jnp.float32)
    m23 = (idx != 1.0).astype(jnp.float32)
    m3 = jnp.concatenate([m12, m13, m23], axis=0)
    maskc_ref[...] = jnp.transpose(m3)
    he = he_ref[...]
    sw = _dot(m3, he)
    sq = _dot(m3, he * he)
    s1 = sw[0:1, :EH] + sw[1:2, EH:2 * EH] + sw[2:3, 2 * EH:]
    s2 = sq[0:1, :EH] + sq[1:2, EH:2 * EH] + sq[2:3, 2 * EH:]
    cnt = jnp.sum(m3).reshape(1, 1)
    row0 = jnp.concatenate([s1, cnt, jnp.zeros((1, 7), jnp.float32)], axis=1)
    row1 = jnp.concatenate([s2, jnp.zeros((1, 8), jnp.float32)], axis=1)
    part = jnp.concatenate([row0, row1], axis=0)

    @pl.when(i == 0)
    def _():
        est_ref[...] = jnp.zeros_like(est_ref)

    est_ref[...] += part


def _k4(y_ref, st_ref, g0_ref, be0_ref, he_ref, maskc_ref, est_ref,
        ge_ref, bee_ref, we2_ref, be2_ref, out_ref, *, n3):
    c = pl.program_id(1)
    tm = y_ref.shape[0]

    @pl.when(c == 0)
    def _():
        xs = _bn_cols(y_ref[...], st_ref[...], g0_ref[...], be0_ref[...],
                      n3, 0, 128)
        for k in range(3):
            out_ref[pl.Slice(k, tm, 3), :] = xs[k]

    @pl.when(c == 1)
    def _():
        xs = _bn_cols(y_ref[...], st_ref[...], g0_ref[...], be0_ref[...],
                      n3, 128, D)
        for k in range(3):
            out_ref[pl.Slice(k, tm, 3), :] = xs[k]

    @pl.when(c == 2)
    def _():
        est = est_ref[...]
        cnt = est[0:1, EH:EH + 1]
        mu = est[0:1, :EH] / cnt
        var = est[1:2, :EH] / cnt - mu * mu
        rs = jax.lax.rsqrt(var + EPS)
        ge, bee = ge_ref[...], bee_ref[...]
        we2, be2 = we2_ref[...], be2_ref[...]
        he = he_ref[...]
        maskc = maskc_ref[...]
        ea = []
        for e in range(3):
            h = he[:, e * EH:(e + 1) * EH]
            eh = _leaky(ge * (h - mu) * rs + bee)
            ea.append(_leaky(_dot(eh, we2) + be2) * maskc[:, e:e + 1])
        ea12, ea13, ea23 = ea
        agg = [ea12 + ea13, ea12 + ea23, ea13 + ea23]
        pad = jnp.zeros((tm, 128 - EH), jnp.float32)
        for k in range(3):
            out_ref[pl.Slice(k, tm, 3), :] = jnp.concatenate(
                [agg[k], pad], axis=1)


def kernel(z, W0, b0, g0, be0, Wl1, bl1, gl, bel, Wl2, bl2, We1, be1, ge,
           bee, We2, be2):
    n = z.shape[0]
    in_dim = z.shape[1]
    n3 = float(3 * n)
    nf = float(n)
    gum_t = jnp.transpose(
        jax.random.gumbel(jax.random.key(1), (n, 4), jnp.float32))
    b0r = b0.reshape(1, -1)
    g0r, be0r = g0.reshape(1, -1), be0.reshape(1, -1)
    bl1r = bl1.reshape(1, -1)
    gl3 = jnp.tile(gl, 3).reshape(1, -1)
    bel3 = jnp.tile(bel, 3).reshape(1, -1)
    wl2x = jnp.zeros((D3, 8), jnp.float32)
    for l in range(3):
        wl2x = wl2x.at[l * D:(l + 1) * D, 2 * l:2 * l + 2].set(Wl2)
    wl2x = wl2x.astype(jnp.bfloat16)
    bl2x = jnp.concatenate([jnp.tile(bl2, 3), jnp.zeros(2, jnp.float32)]
                           ).reshape(1, -1)
    be1r = be1.reshape(1, -1)
    ger, beer = ge.reshape(1, -1), bee.reshape(1, -1)
    be2r = be2.reshape(1, -1)
    grid = (n // TM,)
    rep = lambda i: (0, 0)
    row = lambda i: (i, 0)

    y, st = pl.pallas_call(
        _k1,
        grid=grid,
        in_specs=[pl.BlockSpec((TM, in_dim), row),
                  pl.BlockSpec((in_dim, D3), rep),
                  pl.BlockSpec((1, D3), rep)],
        out_specs=[pl.BlockSpec((TM, D3), row),
                   pl.BlockSpec((2, D), rep)],
        out_shape=[jax.ShapeDtypeStruct((n, D3), jnp.bfloat16),
                   jax.ShapeDtypeStruct((2, D), jnp.float32)],
    )(z, W0.astype(jnp.bfloat16), b0r)

    hpre, he, lst = pl.pallas_call(
        functools.partial(_k2, n3=n3),
        grid=grid,
        in_specs=[pl.BlockSpec((TM, D3), row),
                  pl.BlockSpec((2, D), rep),
                  pl.BlockSpec((1, D), rep),
                  pl.BlockSpec((1, D), rep),
                  pl.BlockSpec((2 * D, D), rep),
                  pl.BlockSpec((1, D), rep),
                  pl.BlockSpec((2 * D, EH), rep),
                  pl.BlockSpec((1, EH), rep)],
        out_specs=[pl.BlockSpec((TM, D3), row),
                   pl.BlockSpec((TM, 3 * EH), row),
                   pl.BlockSpec((2, D3), rep)],
        out_shape=[jax.ShapeDtypeStruct((n, D3), jnp.bfloat16),
                   jax.ShapeDtypeStruct((n, 3 * EH), jnp.float32),
                   jax.ShapeDtypeStruct((2, D3), jnp.float32)],
    )(y, st, g0r, be0r, Wl1.astype(jnp.bfloat16), bl1r,
      We1.astype(jnp.bfloat16), be1r)

    maskc, est = pl.pallas_call(
        functools.partial(_k3, n=nf),
        grid=grid,
        in_specs=[pl.BlockSpec((TM, D3), row),
                  pl.BlockSpec((2, D3), rep),
                  pl.BlockSpec((1, D3), rep),
                  pl.BlockSpec((1, D3), rep),
                  pl.BlockSpec((D3, 8), rep),
                  pl.BlockSpec((1, 8), rep),
                  pl.BlockSpec((4, TM), lambda i: (0, i)),
                  pl.BlockSpec((TM, 3 * EH), row)],
        out_specs=[pl.BlockSpec((TM, 3), row),
                   pl.BlockSpec((2, 2 * EH), rep)],
        out_shape=[jax.ShapeDtypeStruct((n, 3), jnp.float32),
                   jax.ShapeDtypeStruct((2, 2 * EH), jnp.float32)],
    )(hpre, lst, gl3, bel3, wl2x, bl2x, gum_t, he)

    row2 = lambda i, c: (i, 0)
    rep2 = lambda i, c: (0, 0)
    out = pl.pallas_call(
        functools.partial(_k4, n3=n3),
        grid=(n // TM, 3),
        in_specs=[pl.BlockSpec((TM, D3), row2),
                  pl.BlockSpec((2, D), rep2),
                  pl.BlockSpec((1, D), rep2),
                  pl.BlockSpec((1, D), rep2),
                  pl.BlockSpec((TM, 3 * EH), row2),
                  pl.BlockSpec((TM, 3), row2),
                  pl.BlockSpec((2, 2 * EH), rep2),
                  pl.BlockSpec((1, EH), rep2),
                  pl.BlockSpec((1, EH), rep2),
                  pl.BlockSpec((EH, EH), rep2),
                  pl.BlockSpec((1, EH), rep2)],
        out_specs=pl.BlockSpec((3 * TM, 128), lambda i, c: (i, c)),
        out_shape=jax.ShapeDtypeStruct((3 * n, D + EH), jnp.float32),
    )(y, st, g0r, be0r, he, maskc, est, ger, beer, We2, be2r)

    return out


# speedup vs baseline: 5.2218x; 1.2205x over previous
import functools

import jax
import jax.numpy as jnp
import numpy as np
from jax.experimental import pallas as pl

D = 256
D3 = 3 * D
EH = 8
LEAK = 0.05
EPS = 1e-5
TM = 1024

def _leaky(x):
    return jnp.where(x >= 0, x, LEAK * x)


def _dot(a, b):
    return jnp.dot(a, b, preferred_element_type=jnp.float32)


def _bdot(a, b):
    return jnp.dot(a.astype(jnp.bfloat16), b,
                   preferred_element_type=jnp.float32)


def _k1(z_ref, w0_ref, b0_ref, y_ref, st_ref):
    i = pl.program_id(0)
    zb = z_ref[...].astype(jnp.bfloat16)
    y = _leaky(_dot(zb, w0_ref[...]) + b0_ref[...])
    y_ref[...] = y.astype(jnp.bfloat16)
    y0, y1, y2 = y[:, :D], y[:, D:2 * D], y[:, 2 * D:]
    s1 = jnp.sum(y0 + y1 + y2, axis=0, keepdims=True)
    s2 = jnp.sum(y0 * y0 + y1 * y1 + y2 * y2, axis=0, keepdims=True)
    part = jnp.concatenate([s1, s2], axis=0)

    @pl.when(i == 0)
    def _():
        st_ref[...] = jnp.zeros_like(st_ref)

    st_ref[...] += part


def _bn_cols(y, st, g0, be0, n3, lo, hi):
    mu = st[0:1, lo:hi] / n3
    var = st[1:2, lo:hi] / n3 - mu * mu
    rs = jax.lax.rsqrt(var + EPS)
    g, b = g0[:, lo:hi], be0[:, lo:hi]
    out = []
    for k in range(3):
        xk = y[:, k * D + lo:k * D + hi].astype(jnp.float32)
        out.append(_leaky(g * (xk - mu) * rs + b))
    return out


def _k2(y_ref, st_ref, g0_ref, be0_ref, wl1_ref, bl1_ref, we1_ref, be1_ref,
        hpre_ref, he_ref, lst_ref, *, n3):
    i = pl.program_id(0)
    xs = _bn_cols(y_ref[...], st_ref[...], g0_ref[...], be0_ref[...],
                  n3, 0, D)
    wt, wb = wl1_ref[:D, :], wl1_ref[D:, :]
    a0 = _bdot(xs[0], wt)
    a1 = _bdot(xs[1], wt)
    b1 = _bdot(xs[1], wb)
    b2 = _bdot(xs[2], wb)
    bl1 = bl1_ref[...]
    hpre = jnp.concatenate([a0 + b1 + bl1, a1 + b2 + bl1, a0 + b2 + bl1],
                           axis=1)
    hpre_ref[...] = hpre.astype(jnp.bfloat16)
    et, eb = we1_ref[:D, :], we1_ref[D:, :]
    ap0 = _bdot(xs[0], et)
    ap1 = _bdot(xs[1], et)
    bp1 = _bdot(xs[1], eb)
    bp2 = _bdot(xs[2], eb)
    be1 = be1_ref[...]
    he_ref[...] = jnp.concatenate(
        [ap0 + bp1 + be1, ap0 + bp2 + be1, ap1 + bp2 + be1], axis=1)
    s1 = jnp.sum(hpre, axis=0, keepdims=True)
    s2 = jnp.sum(hpre * hpre, axis=0, keepdims=True)
    part = jnp.concatenate([s1, s2], axis=0)

    @pl.when(i == 0)
    def _():
        lst_ref[...] = jnp.zeros_like(lst_ref)

    lst_ref[...] += part


def _k3(hpre_ref, lst_ref, gl3_ref, bel3_ref, wl2x_ref, bl2x_ref, gum_ref,
        he_ref, maskc_ref, est_ref, *, n):
    i = pl.program_id(0)
    st = lst_ref[...]
    mu = st[0:1] / n
    var = st[1:2] / n - mu * mu
    rs = jax.lax.rsqrt(var + EPS)
    hl = hpre_ref[...].astype(jnp.float32)
    h = _leaky(gl3_ref[...] * (hl - mu) * rs + bel3_ref[...])
    a8 = _leaky(_bdot(h, wl2x_ref[...]) + bl2x_ref[...])
    at = jnp.transpose(a8)
    r = [at[j:j + 1, :] for j in range(6)]
    s135 = r[1] + r[3] + r[5]
    ep0 = (r[0] + r[3] + r[5]) / 3
    ep1 = (r[2] + r[1] + r[5]) / 3
    ep2 = (r[4] + r[3] + r[1]) / 3
    ep3 = s135 / 3
    m = jnp.maximum(jnp.maximum(ep0, ep1), jnp.maximum(ep2, ep3))
    u0 = jnp.exp(ep0 - m)
    u1 = jnp.exp(ep1 - m)
    u2 = jnp.exp(ep2 - m)
    u3 = jnp.exp(ep3 - m)
    s = u0 + u1 + u2 + u3
    gum = gum_ref[...]
    l0 = jnp.log(u0 / s + 1e-4) + gum[0:1, :]
    l1 = jnp.log(u1 / s + 1e-4) + gum[1:2, :]
    l2 = jnp.log(u2 / s + 1e-4) + gum[2:3, :]
    l3 = jnp.log(u3 / s + 1e-4) + gum[3:4, :]
    best = l0
    idx = jnp.zeros_like(l0)
    for j, lj in ((1, l1), (2, l2), (3, l3)):
        gt = lj > best
        best = jnp.where(gt, lj, best)
        idx = jnp.where(gt, jnp.float32(j), idx)
    m12 = (idx != 0.0).astype(jnp.float32)
    m13 = (idx != 2.0).astype(jnp.float32)
    m23 = (idx != 1.0).astype(jnp.float32)
    m3 = jnp.concatenate([m12, m13, m23], axis=0)
    maskc_ref[...] = jnp.transpose(m3)
    he = he_ref[...]
    sw = _dot(m3, he)
    sq = _dot(m3, he * he)
    s1 = sw[0:1, :EH] + sw[1:2, EH:2 * EH] + sw[2:3, 2 * EH:]
    s2 = sq[0:1, :EH] + sq[1:2, EH:2 * EH] + sq[2:3, 2 * EH:]
    cnt = jnp.sum(m3).reshape(1, 1)
    row0 = jnp.concatenate([s1, cnt, jnp.zeros((1, 7), jnp.float32)], axis=1)
    row1 = jnp.concatenate([s2, jnp.zeros((1, 8), jnp.float32)], axis=1)
    part = jnp.concatenate([row0, row1], axis=0)

    @pl.when(i == 0)
    def _():
        est_ref[...] = jnp.zeros_like(est_ref)

    est_ref[...] += part


def _k4(y_ref, st_ref, g0_ref, be0_ref, he_ref, maskc_ref, est_ref,
        ge_ref, bee_ref, we2_ref, be2_ref, out_ref, *, n3):
    c = pl.program_id(1)
    tm = y_ref.shape[0]

    @pl.when(c == 0)
    def _():
        xs = _bn_cols(y_ref[...], st_ref[...], g0_ref[...], be0_ref[...],
                      n3, 0, 128)
        for k in range(3):
            out_ref[pl.Slice(k, tm, 3), :] = xs[k]

    @pl.when(c == 1)
    def _():
        xs = _bn_cols(y_ref[...], st_ref[...], g0_ref[...], be0_ref[...],
                      n3, 128, D)
        for k in range(3):
            out_ref[pl.Slice(k, tm, 3), :] = xs[k]

    @pl.when(c == 2)
    def _():
        est = est_ref[...]
        cnt = est[0:1, EH:EH + 1]
        mu = est[0:1, :EH] / cnt
        var = est[1:2, :EH] / cnt - mu * mu
        rs = jax.lax.rsqrt(var + EPS)
        ge, bee = ge_ref[...], bee_ref[...]
        we2, be2 = we2_ref[...], be2_ref[...]
        he = he_ref[...]
        maskc = maskc_ref[...]
        ea = []
        for e in range(3):
            h = he[:, e * EH:(e + 1) * EH]
            eh = _leaky(ge * (h - mu) * rs + bee)
            ea.append(_leaky(_dot(eh, we2) + be2) * maskc[:, e:e + 1])
        ea12, ea13, ea23 = ea
        agg = [ea12 + ea13, ea12 + ea23, ea13 + ea23]
        pad = jnp.zeros((tm, 128 - EH), jnp.float32)
        for k in range(3):
            out_ref[pl.Slice(k, tm, 3), :] = jnp.concatenate(
                [agg[k], pad], axis=1)


def kernel(z, W0, b0, g0, be0, Wl1, bl1, gl, bel, Wl2, bl2, We1, be1, ge,
           bee, We2, be2):
    n = z.shape[0]
    in_dim = z.shape[1]
    n3 = float(3 * n)
    nf = float(n)
    gum_t = jnp.transpose(
        jax.random.gumbel(jax.random.key(1), (n, 4), jnp.float32))
    b0r = b0.reshape(1, -1)
    g0r, be0r = g0.reshape(1, -1), be0.reshape(1, -1)
    bl1r = bl1.reshape(1, -1)
    gl3 = jnp.tile(gl, 3).reshape(1, -1)
    bel3 = jnp.tile(bel, 3).reshape(1, -1)
    wl2x = jnp.zeros((D3, 8), jnp.float32)
    for l in range(3):
        wl2x = wl2x.at[l * D:(l + 1) * D, 2 * l:2 * l + 2].set(Wl2)
    wl2x = wl2x.astype(jnp.bfloat16)
    bl2x = jnp.concatenate([jnp.tile(bl2, 3), jnp.zeros(2, jnp.float32)]
                           ).reshape(1, -1)
    be1r = be1.reshape(1, -1)
    ger, beer = ge.reshape(1, -1), bee.reshape(1, -1)
    be2r = be2.reshape(1, -1)
    grid = (n // TM,)
    rep = lambda i: (0, 0)
    row = lambda i: (i, 0)

    y, st = pl.pallas_call(
        _k1,
        grid=grid,
        in_specs=[pl.BlockSpec((TM, in_dim), row),
                  pl.BlockSpec((in_dim, D3), rep),
                  pl.BlockSpec((1, D3), rep)],
        out_specs=[pl.BlockSpec((TM, D3), row),
                   pl.BlockSpec((2, D), rep)],
        out_shape=[jax.ShapeDtypeStruct((n, D3), jnp.bfloat16),
                   jax.ShapeDtypeStruct((2, D), jnp.float32)],
    )(z, W0.astype(jnp.bfloat16), b0r)

    hpre, he, lst = pl.pallas_call(
        functools.partial(_k2, n3=n3),
        grid=grid,
        in_specs=[pl.BlockSpec((TM, D3), row),
                  pl.BlockSpec((2, D), rep),
                  pl.BlockSpec((1, D), rep),
                  pl.BlockSpec((1, D), rep),
                  pl.BlockSpec((2 * D, D), rep),
                  pl.BlockSpec((1, D), rep),
                  pl.BlockSpec((2 * D, EH), rep),
                  pl.BlockSpec((1, EH), rep)],
        out_specs=[pl.BlockSpec((TM, D3), row),
                   pl.BlockSpec((TM, 3 * EH), row),
                   pl.BlockSpec((2, D3), rep)],
        out_shape=[jax.ShapeDtypeStruct((n, D3), jnp.bfloat16),
                   jax.ShapeDtypeStruct((n, 3 * EH), jnp.float32),
                   jax.ShapeDtypeStruct((2, D3), jnp.float32)],
    )(y, st, g0r, be0r, Wl1.astype(jnp.bfloat16), bl1r,
      We1.astype(jnp.bfloat16), be1r)

    maskc, est = pl.pallas_call(
        functools.partial(_k3, n=nf),
        grid=grid,
        in_specs=[pl.BlockSpec((TM, D3), row),
                  pl.BlockSpec((2, D3), rep),
                  pl.BlockSpec((1, D3), rep),
                  pl.BlockSpec((1, D3), rep),
                  pl.BlockSpec((D3, 8), rep),
                  pl.BlockSpec((1, 8), rep),
                  pl.BlockSpec((4, TM), lambda i: (0, i)),
                  pl.BlockSpec((TM, 3 * EH), row)],
        out_specs=[pl.BlockSpec((TM, 3), row),
                   pl.BlockSpec((2, 2 * EH), rep)],
        out_shape=[jax.ShapeDtypeStruct((n, 3), jnp.float32),
                   jax.ShapeDtypeStruct((2, 2 * EH), jnp.float32)],
    )(hpre, lst, gl3, bel3, wl2x, bl2x, gum_t, he)

    row2 = lambda i, c: (i, 0)
    rep2 = lambda i, c: (0, 0)
    out = pl.pallas_call(
        functools.partial(_k4, n3=n3),
        grid=(n // TM, 3),
        in_specs=[pl.BlockSpec((TM, D3), row2),
                  pl.BlockSpec((2, D), rep2),
                  pl.BlockSpec((1, D), rep2),
                  pl.BlockSpec((1, D), rep2),
                  pl.BlockSpec((TM, 3 * EH), row2),
                  pl.BlockSpec((TM, 3), row2),
                  pl.BlockSpec((2, 2 * EH), rep2),
                  pl.BlockSpec((1, EH), rep2),
                  pl.BlockSpec((1, EH), rep2),
                  pl.BlockSpec((EH, EH), rep2),
                  pl.BlockSpec((1, EH), rep2)],
        out_specs=pl.BlockSpec((3 * TM, 128), lambda i, c: (i, c)),
        out_shape=jax.ShapeDtypeStruct((3 * n, D + EH), jnp.float32),
    )(y, st, g0r, be0r, he, maskc, est, ger, beer, We2, be2r)

    return out


# speedup vs baseline: 5.6156x; 1.0754x over previous
import functools

import jax
import jax.numpy as jnp
import numpy as np
from jax.experimental import pallas as pl

D = 256
D3 = 3 * D
EH = 8
LEAK = 0.05
EPS = 1e-5
TM = 2048

def _leaky(x):
    return jnp.where(x >= 0, x, LEAK * x)


def _dot(a, b):
    return jnp.dot(a, b, preferred_element_type=jnp.float32)


def _bdot(a, b):
    return jnp.dot(a.astype(jnp.bfloat16), b,
                   preferred_element_type=jnp.float32)


def _k1(z_ref, w0_ref, b0_ref, y_ref, st_ref):
    i = pl.program_id(0)
    zb = z_ref[...].astype(jnp.bfloat16)
    y = _leaky(_dot(zb, w0_ref[...]) + b0_ref[...])
    y_ref[...] = y.astype(jnp.bfloat16)
    y0, y1, y2 = y[:, :D], y[:, D:2 * D], y[:, 2 * D:]
    s1 = jnp.sum(y0 + y1 + y2, axis=0, keepdims=True)
    s2 = jnp.sum(y0 * y0 + y1 * y1 + y2 * y2, axis=0, keepdims=True)
    part = jnp.concatenate([s1, s2], axis=0)

    @pl.when(i == 0)
    def _():
        st_ref[...] = jnp.zeros_like(st_ref)

    st_ref[...] += part


def _bn_cols(y, st, g0, be0, n3, lo, hi):
    mu = st[0:1, lo:hi] / n3
    var = st[1:2, lo:hi] / n3 - mu * mu
    rs = jax.lax.rsqrt(var + EPS)
    g, b = g0[:, lo:hi], be0[:, lo:hi]
    out = []
    for k in range(3):
        xk = y[:, k * D + lo:k * D + hi].astype(jnp.float32)
        out.append(_leaky(g * (xk - mu) * rs + b))
    return out


def _k2(y_ref, st_ref, g0_ref, be0_ref, wl1_ref, bl1_ref, we1_ref, be1_ref,
        hpre_ref, he_ref, lst_ref, *, n3):
    i = pl.program_id(0)
    xs = _bn_cols(y_ref[...], st_ref[...], g0_ref[...], be0_ref[...],
                  n3, 0, D)
    wt, wb = wl1_ref[:D, :], wl1_ref[D:, :]
    a0 = _bdot(xs[0], wt)
    a1 = _bdot(xs[1], wt)
    b1 = _bdot(xs[1], wb)
    b2 = _bdot(xs[2], wb)
    bl1 = bl1_ref[...]
    hpre = jnp.concatenate([a0 + b1 + bl1, a1 + b2 + bl1, a0 + b2 + bl1],
                           axis=1)
    hpre_ref[...] = hpre.astype(jnp.bfloat16)
    et, eb = we1_ref[:D, :], we1_ref[D:, :]
    ap0 = _bdot(xs[0], et)
    ap1 = _bdot(xs[1], et)
    bp1 = _bdot(xs[1], eb)
    bp2 = _bdot(xs[2], eb)
    be1 = be1_ref[...]
    he_ref[...] = jnp.concatenate(
        [ap0 + bp1 + be1, ap0 + bp2 + be1, ap1 + bp2 + be1], axis=1)
    s1 = jnp.sum(hpre, axis=0, keepdims=True)
    s2 = jnp.sum(hpre * hpre, axis=0, keepdims=True)
    part = jnp.concatenate([s1, s2], axis=0)

    @pl.when(i == 0)
    def _():
        lst_ref[...] = jnp.zeros_like(lst_ref)

    lst_ref[...] += part


def _k3(hpre_ref, lst_ref, gl3_ref, bel3_ref, wl2x_ref, bl2x_ref, gum_ref,
        he_ref, maskc_ref, est_ref, *, n):
    i = pl.program_id(0)
    st = lst_ref[...]
    mu = st[0:1] / n
    var = st[1:2] / n - mu * mu
    rs = jax.lax.rsqrt(var + EPS)
    hl = hpre_ref[...].astype(jnp.float32)
    h = _leaky(gl3_ref[...] * (hl - mu) * rs + bel3_ref[...])
    a8 = _leaky(_bdot(h, wl2x_ref[...]) + bl2x_ref[...])
    at = jnp.transpose(a8)
    r = [at[j:j + 1, :] for j in range(6)]
    s135 = r[1] + r[3] + r[5]
    ep0 = (r[0] + r[3] + r[5]) / 3
    ep1 = (r[2] + r[1] + r[5]) / 3
    ep2 = (r[4] + r[3] + r[1]) / 3
    ep3 = s135 / 3
    m = jnp.maximum(jnp.maximum(ep0, ep1), jnp.maximum(ep2, ep3))
    u0 = jnp.exp(ep0 - m)
    u1 = jnp.exp(ep1 - m)
    u2 = jnp.exp(ep2 - m)
    u3 = jnp.exp(ep3 - m)
    s = u0 + u1 + u2 + u3
    gum = gum_ref[...]
    l0 = jnp.log(u0 / s + 1e-4) + gum[0:1, :]
    l1 = jnp.log(u1 / s + 1e-4) + gum[1:2, :]
    l2 = jnp.log(u2 / s + 1e-4) + gum[2:3, :]
    l3 = jnp.log(u3 / s + 1e-4) + gum[3:4, :]
    best = l0
    idx = jnp.zeros_like(l0)
    for j, lj in ((1, l1), (2, l2), (3, l3)):
        gt = lj > best
        best = jnp.where(gt, lj, best)
        idx = jnp.where(gt, jnp.float32(j), idx)
    m12 = (idx != 0.0).astype(jnp.float32)
    m13 = (idx != 2.0).astype(jnp.float32)
    m23 = (idx != 1.0).astype(jnp.float32)
    m3 = jnp.concatenate([m12, m13, m23], axis=0)
    maskc_ref[...] = jnp.transpose(m3)
    he = he_ref[...]
    sw = _dot(m3, he)
    sq = _dot(m3, he * he)
    s1 = sw[0:1, :EH] + sw[1:2, EH:2 * EH] + sw[2:3, 2 * EH:]
    s2 = sq[0:1, :EH] + sq[1:2, EH:2 * EH] + sq[2:3, 2 * EH:]
    cnt = jnp.sum(m3).reshape(1, 1)
    row0 = jnp.concatenate([s1, cnt, jnp.zeros((1, 7), jnp.float32)], axis=1)
    row1 = jnp.concatenate([s2, jnp.zeros((1, 8), jnp.float32)], axis=1)
    part = jnp.concatenate([row0, row1], axis=0)

    @pl.when(i == 0)
    def _():
        est_ref[...] = jnp.zeros_like(est_ref)

    est_ref[...] += part


def _k4(y_ref, st_ref, g0_ref, be0_ref, he_ref, maskc_ref, est_ref,
        ge_ref, bee_ref, we2_ref, be2_ref, out_ref, *, n3):
    c = pl.program_id(1)
    tm = y_ref.shape[0]

    @pl.when(c == 0)
    def _():
        xs = _bn_cols(y_ref[...], st_ref[...], g0_ref[...], be0_ref[...],
                      n3, 0, 128)
        for k in range(3):
            out_ref[pl.Slice(k, tm, 3), :] = xs[k]

    @pl.when(c == 1)
    def _():
        xs = _bn_cols(y_ref[...], st_ref[...], g0_ref[...], be0_ref[...],
                      n3, 128, D)
        for k in range(3):
            out_ref[pl.Slice(k, tm, 3), :] = xs[k]

    @pl.when(c == 2)
    def _():
        est = est_ref[...]
        cnt = est[0:1, EH:EH + 1]
        mu = est[0:1, :EH] / cnt
        var = est[1:2, :EH] / cnt - mu * mu
        rs = jax.lax.rsqrt(var + EPS)
        ge, bee = ge_ref[...], bee_ref[...]
        we2, be2 = we2_ref[...], be2_ref[...]
        he = he_ref[...]
        maskc = maskc_ref[...]
        ea = []
        for e in range(3):
            h = he[:, e * EH:(e + 1) * EH]
            eh = _leaky(ge * (h - mu) * rs + bee)
            ea.append(_leaky(_dot(eh, we2) + be2) * maskc[:, e:e + 1])
        ea12, ea13, ea23 = ea
        agg = [ea12 + ea13, ea12 + ea23, ea13 + ea23]
        pad = jnp.zeros((tm, 128 - EH), jnp.float32)
        for k in range(3):
            out_ref[pl.Slice(k, tm, 3), :] = jnp.concatenate(
                [agg[k], pad], axis=1)


def kernel(z, W0, b0, g0, be0, Wl1, bl1, gl, bel, Wl2, bl2, We1, be1, ge,
           bee, We2, be2):
    n = z.shape[0]
    in_dim = z.shape[1]
    n3 = float(3 * n)
    nf = float(n)
    gum_t = jnp.transpose(
        jax.random.gumbel(jax.random.key(1), (n, 4), jnp.float32))
    b0r = b0.reshape(1, -1)
    g0r, be0r = g0.reshape(1, -1), be0.reshape(1, -1)
    bl1r = bl1.reshape(1, -1)
    gl3 = jnp.tile(gl, 3).reshape(1, -1)
    bel3 = jnp.tile(bel, 3).reshape(1, -1)
    wl2x = jnp.zeros((D3, 8), jnp.float32)
    for l in range(3):
        wl2x = wl2x.at[l * D:(l + 1) * D, 2 * l:2 * l + 2].set(Wl2)
    wl2x = wl2x.astype(jnp.bfloat16)
    bl2x = jnp.concatenate([jnp.tile(bl2, 3), jnp.zeros(2, jnp.float32)]
                           ).reshape(1, -1)
    be1r = be1.reshape(1, -1)
    ger, beer = ge.reshape(1, -1), bee.reshape(1, -1)
    be2r = be2.reshape(1, -1)
    grid = (n // TM,)
    rep = lambda i: (0, 0)
    row = lambda i: (i, 0)

    y, st = pl.pallas_call(
        _k1,
        grid=grid,
        in_specs=[pl.BlockSpec((TM, in_dim), row),
                  pl.BlockSpec((in_dim, D3), rep),
                  pl.BlockSpec((1, D3), rep)],
        out_specs=[pl.BlockSpec((TM, D3), row),
                   pl.BlockSpec((2, D), rep)],
        out_shape=[jax.ShapeDtypeStruct((n, D3), jnp.bfloat16),
                   jax.ShapeDtypeStruct((2, D), jnp.float32)],
    )(z, W0.astype(jnp.bfloat16), b0r)

    hpre, he, lst = pl.pallas_call(
        functools.partial(_k2, n3=n3),
        grid=grid,
        in_specs=[pl.BlockSpec((TM, D3), row),
                  pl.BlockSpec((2, D), rep),
                  pl.BlockSpec((1, D), rep),
                  pl.BlockSpec((1, D), rep),
                  pl.BlockSpec((2 * D, D), rep),
                  pl.BlockSpec((1, D), rep),
                  pl.BlockSpec((2 * D, EH), rep),
                  pl.BlockSpec((1, EH), rep)],
        out_specs=[pl.BlockSpec((TM, D3), row),
                   pl.BlockSpec((TM, 3 * EH), row),
                   pl.BlockSpec((2, D3), rep)],
        out_shape=[jax.ShapeDtypeStruct((n, D3), jnp.bfloat16),
                   jax.ShapeDtypeStruct((n, 3 * EH), jnp.float32),
                   jax.ShapeDtypeStruct((2, D3), jnp.float32)],
    )(y, st, g0r, be0r, Wl1.astype(jnp.bfloat16), bl1r,
      We1.astype(jnp.bfloat16), be1r)

    maskc, est = pl.pallas_call(
        functools.partial(_k3, n=nf),
        grid=grid,
        in_specs=[pl.BlockSpec((TM, D3), row),
                  pl.BlockSpec((2, D3), rep),
                  pl.BlockSpec((1, D3), rep),
                  pl.BlockSpec((1, D3), rep),
                  pl.BlockSpec((D3, 8), rep),
                  pl.BlockSpec((1, 8), rep),
                  pl.BlockSpec((4, TM), lambda i: (0, i)),
                  pl.BlockSpec((TM, 3 * EH), row)],
        out_specs=[pl.BlockSpec((TM, 3), row),
                   pl.BlockSpec((2, 2 * EH), rep)],
        out_shape=[jax.ShapeDtypeStruct((n, 3), jnp.float32),
                   jax.ShapeDtypeStruct((2, 2 * EH), jnp.float32)],
    )(hpre, lst, gl3, bel3, wl2x, bl2x, gum_t, he)

    row2 = lambda i, c: (i, 0)
    rep2 = lambda i, c: (0, 0)
    out = pl.pallas_call(
        functools.partial(_k4, n3=n3),
        grid=(n // TM, 3),
        in_specs=[pl.BlockSpec((TM, D3), row2),
                  pl.BlockSpec((2, D), rep2),
                  pl.BlockSpec((1, D), rep2),
                  pl.BlockSpec((1, D), rep2),
                  pl.BlockSpec((TM, 3 * EH), row2),
                  pl.BlockSpec((TM, 3), row2),
                  pl.BlockSpec((2, 2 * EH), rep2),
                  pl.BlockSpec((1, EH), rep2),
                  pl.BlockSpec((1, EH), rep2),
                  pl.BlockSpec((EH, EH), rep2),
                  pl.BlockSpec((1, EH), rep2)],
        out_specs=pl.BlockSpec((3 * TM, 128), lambda i, c: (i, c)),
        out_shape=jax.ShapeDtypeStruct((3 * n, D + EH), jnp.float32),
    )(y, st, g0r, be0r, he, maskc, est, ger, beer, We2, be2r)

    return out


# speedup vs baseline: 5.9101x; 1.0524x over previous
import functools

import jax
import jax.numpy as jnp
import numpy as np
from jax.experimental import pallas as pl

D = 256
D3 = 3 * D
EH = 8
LEAK = 0.05
EPS = 1e-5
TM1 = 4096
TM = 2048
TM3 = 4096

def _leaky(x):
    return jnp.where(x >= 0, x, LEAK * x)


def _dot(a, b):
    return jnp.dot(a, b, preferred_element_type=jnp.float32)


def _bdot(a, b):
    return jnp.dot(a.astype(jnp.bfloat16), b,
                   preferred_element_type=jnp.float32)


def _k1(z_ref, w0_ref, b0_ref, y_ref, st_ref):
    i = pl.program_id(0)
    zb = z_ref[...].astype(jnp.bfloat16)
    y = _leaky(_dot(zb, w0_ref[...]) + b0_ref[...])
    y_ref[...] = y.astype(jnp.bfloat16)
    y0, y1, y2 = y[:, :D], y[:, D:2 * D], y[:, 2 * D:]
    s1 = jnp.sum(y0 + y1 + y2, axis=0, keepdims=True)
    s2 = jnp.sum(y0 * y0 + y1 * y1 + y2 * y2, axis=0, keepdims=True)
    part = jnp.concatenate([s1, s2], axis=0)

    @pl.when(i == 0)
    def _():
        st_ref[...] = jnp.zeros_like(st_ref)

    st_ref[...] += part


def _bn_cols(y, st, g0, be0, n3, lo, hi):
    mu = st[0:1, lo:hi] / n3
    var = st[1:2, lo:hi] / n3 - mu * mu
    rs = jax.lax.rsqrt(var + EPS)
    g, b = g0[:, lo:hi], be0[:, lo:hi]
    out = []
    for k in range(3):
        xk = y[:, k * D + lo:k * D + hi].astype(jnp.float32)
        out.append(_leaky(g * (xk - mu) * rs + b))
    return out


def _k2(y_ref, st_ref, g0_ref, be0_ref, wl1_ref, bl1_ref, we1_ref, be1_ref,
        hpre_ref, he_ref, lst_ref, *, n3):
    i = pl.program_id(0)
    xs = _bn_cols(y_ref[...], st_ref[...], g0_ref[...], be0_ref[...],
                  n3, 0, D)
    wt, wb = wl1_ref[:D, :], wl1_ref[D:, :]
    a0 = _bdot(xs[0], wt)
    a1 = _bdot(xs[1], wt)
    b1 = _bdot(xs[1], wb)
    b2 = _bdot(xs[2], wb)
    bl1 = bl1_ref[...]
    hpre = jnp.concatenate([a0 + b1 + bl1, a1 + b2 + bl1, a0 + b2 + bl1],
                           axis=1)
    hpre_ref[...] = hpre.astype(jnp.bfloat16)
    et, eb = we1_ref[:D, :], we1_ref[D:, :]
    ap0 = _bdot(xs[0], et)
    ap1 = _bdot(xs[1], et)
    bp1 = _bdot(xs[1], eb)
    bp2 = _bdot(xs[2], eb)
    be1 = be1_ref[...]
    he_ref[...] = jnp.concatenate(
        [ap0 + bp1 + be1, ap0 + bp2 + be1, ap1 + bp2 + be1],
        axis=1).astype(jnp.bfloat16)
    s1 = jnp.sum(hpre, axis=0, keepdims=True)
    s2 = jnp.sum(hpre * hpre, axis=0, keepdims=True)
    part = jnp.concatenate([s1, s2], axis=0)

    @pl.when(i == 0)
    def _():
        lst_ref[...] = jnp.zeros_like(lst_ref)

    lst_ref[...] += part


def _k3(hpre_ref, lst_ref, gl_ref, bel_ref, wl2_ref, bl2_ref, gum_ref,
        he_ref, maskc_ref, est_ref, *, n):
    i = pl.program_id(0)
    st = lst_ref[...]
    mu = st[0:1] / n
    var = st[1:2] / n - mu * mu
    rs = jax.lax.rsqrt(var + EPS)
    gl, bel = gl_ref[...], bel_ref[...]
    gl3 = jnp.concatenate([gl, gl, gl], axis=1)
    bel3 = jnp.concatenate([bel, bel, bel], axis=1)
    wl2 = wl2_ref[...]
    z2 = jnp.zeros((D, 2), jnp.float32)
    wl2x = jnp.concatenate([
        jnp.concatenate([wl2, z2, z2, z2], axis=1),
        jnp.concatenate([z2, wl2, z2, z2], axis=1),
        jnp.concatenate([z2, z2, wl2, z2], axis=1)], axis=0)
    bl2 = bl2_ref[...]
    bl2x = jnp.concatenate(
        [bl2, bl2, bl2, jnp.zeros((1, 2), jnp.float32)], axis=1)
    hl = hpre_ref[...].astype(jnp.float32)
    h = _leaky(gl3 * (hl - mu) * rs + bel3)
    a8 = _leaky(_bdot(h, wl2x.astype(jnp.bfloat16)) + bl2x)
    at = jnp.transpose(a8)
    r = [at[j:j + 1, :] for j in range(6)]
    s135 = r[1] + r[3] + r[5]
    ep0 = (r[0] + r[3] + r[5]) / 3
    ep1 = (r[2] + r[1] + r[5]) / 3
    ep2 = (r[4] + r[3] + r[1]) / 3
    ep3 = s135 / 3
    m = jnp.maximum(jnp.maximum(ep0, ep1), jnp.maximum(ep2, ep3))
    u0 = jnp.exp(ep0 - m)
    u1 = jnp.exp(ep1 - m)
    u2 = jnp.exp(ep2 - m)
    u3 = jnp.exp(ep3 - m)
    s = u0 + u1 + u2 + u3
    gum = gum_ref[...]
    l0 = jnp.log(u0 / s + 1e-4) + gum[0:1, :]
    l1 = jnp.log(u1 / s + 1e-4) + gum[1:2, :]
    l2 = jnp.log(u2 / s + 1e-4) + gum[2:3, :]
    l3 = jnp.log(u3 / s + 1e-4) + gum[3:4, :]
    best = l0
    idx = jnp.zeros_like(l0)
    for j, lj in ((1, l1), (2, l2), (3, l3)):
        gt = lj > best
        best = jnp.where(gt, lj, best)
        idx = jnp.where(gt, jnp.float32(j), idx)
    m12 = (idx != 0.0).astype(jnp.float32)
    m13 = (idx != 2.0).astype(jnp.float32)
    m23 = (idx != 1.0).astype(jnp.float32)
    m3 = jnp.concatenate([m12, m13, m23], axis=0)
    maskc_ref[...] = jnp.transpose(m3)
    he = he_ref[...].astype(jnp.float32)
    sw = _dot(m3, he)
    sq = _dot(m3, he * he)
    s1 = sw[0:1, :EH] + sw[1:2, EH:2 * EH] + sw[2:3, 2 * EH:]
    s2 = sq[0:1, :EH] + sq[1:2, EH:2 * EH] + sq[2:3, 2 * EH:]
    cnt = jnp.sum(m3).reshape(1, 1)
    row0 = jnp.concatenate([s1, cnt, jnp.zeros((1, 7), jnp.float32)], axis=1)
    row1 = jnp.concatenate([s2, jnp.zeros((1, 8), jnp.float32)], axis=1)
    part = jnp.concatenate([row0, row1], axis=0)

    @pl.when(i == 0)
    def _():
        est_ref[...] = jnp.zeros_like(est_ref)

    est_ref[...] += part


def _k4(y_ref, st_ref, g0_ref, be0_ref, he_ref, maskc_ref, est_ref,
        ge_ref, bee_ref, we2_ref, be2_ref, out_ref, *, n3):
    c = pl.program_id(1)
    tm = y_ref.shape[0]

    @pl.when(c == 0)
    def _():
        xs = _bn_cols(y_ref[...], st_ref[...], g0_ref[...], be0_ref[...],
                      n3, 0, 128)
        for k in range(3):
            out_ref[pl.Slice(k, tm, 3), :] = xs[k]

    @pl.when(c == 1)
    def _():
        xs = _bn_cols(y_ref[...], st_ref[...], g0_ref[...], be0_ref[...],
                      n3, 128, D)
        for k in range(3):
            out_ref[pl.Slice(k, tm, 3), :] = xs[k]

    @pl.when(c == 2)
    def _():
        est = est_ref[...]
        cnt = est[0:1, EH:EH + 1]
        mu = est[0:1, :EH] / cnt
        var = est[1:2, :EH] / cnt - mu * mu
        rs = jax.lax.rsqrt(var + EPS)
        ge, bee = ge_ref[...], bee_ref[...]
        we2, be2 = we2_ref[...], be2_ref[...]
        he = he_ref[...].astype(jnp.float32)
        maskc = maskc_ref[...]
        ea = []
        for e in range(3):
            h = he[:, e * EH:(e + 1) * EH]
            eh = _leaky(ge * (h - mu) * rs + bee)
            ea.append(_leaky(_dot(eh, we2) + be2) * maskc[:, e:e + 1])
        ea12, ea13, ea23 = ea
        agg = [ea12 + ea13, ea12 + ea23, ea13 + ea23]
        pad = jnp.zeros((tm, 128 - EH), jnp.float32)
        for k in range(3):
            out_ref[pl.Slice(k, tm, 3), :] = jnp.concatenate(
                [agg[k], pad], axis=1)


def kernel(z, W0, b0, g0, be0, Wl1, bl1, gl, bel, Wl2, bl2, We1, be1, ge,
           bee, We2, be2):
    n = z.shape[0]
    in_dim = z.shape[1]
    n3 = float(3 * n)
    nf = float(n)
    gum_t = jnp.transpose(
        jax.random.gumbel(jax.random.key(1), (n, 4), jnp.float32))
    b0r = b0.reshape(1, -1)
    g0r, be0r = g0.reshape(1, -1), be0.reshape(1, -1)
    bl1r = bl1.reshape(1, -1)
    glr, belr = gl.reshape(1, -1), bel.reshape(1, -1)
    bl2r = bl2.reshape(1, -1)
    be1r = be1.reshape(1, -1)
    ger, beer = ge.reshape(1, -1), bee.reshape(1, -1)
    be2r = be2.reshape(1, -1)
    rep = lambda i: (0, 0)
    row = lambda i: (i, 0)

    y, st = pl.pallas_call(
        _k1,
        grid=(n // TM1,),
        in_specs=[pl.BlockSpec((TM1, in_dim), row),
                  pl.BlockSpec((in_dim, D3), rep),
                  pl.BlockSpec((1, D3), rep)],
        out_specs=[pl.BlockSpec((TM1, D3), row),
                   pl.BlockSpec((2, D), rep)],
        out_shape=[jax.ShapeDtypeStruct((n, D3), jnp.bfloat16),
                   jax.ShapeDtypeStruct((2, D), jnp.float32)],
    )(z, W0.astype(jnp.bfloat16), b0r)

    hpre, he, lst = pl.pallas_call(
        functools.partial(_k2, n3=n3),
        grid=(n // TM,),
        in_specs=[pl.BlockSpec((TM, D3), row),
                  pl.BlockSpec((2, D), rep),
                  pl.BlockSpec((1, D), rep),
                  pl.BlockSpec((1, D), rep),
                  pl.BlockSpec((2 * D, D), rep),
                  pl.BlockSpec((1, D), rep),
                  pl.BlockSpec((2 * D, EH), rep),
                  pl.BlockSpec((1, EH), rep)],
        out_specs=[pl.BlockSpec((TM, D3), row),
                   pl.BlockSpec((TM, 3 * EH), row),
                   pl.BlockSpec((2, D3), rep)],
        out_shape=[jax.ShapeDtypeStruct((n, D3), jnp.bfloat16),
                   jax.ShapeDtypeStruct((n, 3 * EH), jnp.bfloat16),
                   jax.ShapeDtypeStruct((2, D3), jnp.float32)],
    )(y, st, g0r, be0r, Wl1.astype(jnp.bfloat16), bl1r,
      We1.astype(jnp.bfloat16), be1r)

    maskc, est = pl.pallas_call(
        functools.partial(_k3, n=nf),
        grid=(n // TM3,),
        in_specs=[pl.BlockSpec((TM3, D3), row),
                  pl.BlockSpec((2, D3), rep),
                  pl.BlockSpec((1, D), rep),
                  pl.BlockSpec((1, D), rep),
                  pl.BlockSpec((D, 2), rep),
                  pl.BlockSpec((1, 2), rep),
                  pl.BlockSpec((4, TM3), lambda i: (0, i)),
                  pl.BlockSpec((TM3, 3 * EH), row)],
        out_specs=[pl.BlockSpec((TM3, 3), row),
                   pl.BlockSpec((2, 2 * EH), rep)],
        out_shape=[jax.ShapeDtypeStruct((n, 3), jnp.float32),
                   jax.ShapeDtypeStruct((2, 2 * EH), jnp.float32)],
    )(hpre, lst, glr, belr, Wl2, bl2r, gum_t, he)

    row2 = lambda i, c: (i, 0)
    rep2 = lambda i, c: (0, 0)
    out = pl.pallas_call(
        functools.partial(_k4, n3=n3),
        grid=(n // TM3, 3),
        in_specs=[pl.BlockSpec((TM3, D3), row2),
                  pl.BlockSpec((2, D), rep2),
                  pl.BlockSpec((1, D), rep2),
                  pl.BlockSpec((1, D), rep2),
                  pl.BlockSpec((TM3, 3 * EH), row2),
                  pl.BlockSpec((TM3, 3), row2),
                  pl.BlockSpec((2, 2 * EH), rep2),
                  pl.BlockSpec((1, EH), rep2),
                  pl.BlockSpec((1, EH), rep2),
                  pl.BlockSpec((EH, EH), rep2),
                  pl.BlockSpec((1, EH), rep2)],
        out_specs=pl.BlockSpec((3 * TM3, 128), lambda i, c: (i, c)),
        out_shape=jax.ShapeDtypeStruct((3 * n, D + EH), jnp.float32),
    )(y, st, g0r, be0r, he, maskc, est, ger, beer, We2, be2r)

    return out
